# Initial kernel scaffold; baseline (speedup 1.0000x reference)
import jax
import jax.numpy as jnp
from jax.experimental import pallas as pl


def kernel(x, edge_index, W0, b0, W1, b1):
    raise NotImplementedError("write your pallas kernel here")


# speedup vs baseline: 4.8424x; 4.8424x over previous
import functools

import jax
import jax.numpy as jnp
from jax import lax
from jax.experimental import pallas as pl
from jax.experimental.pallas import tpu as pltpu
from jax.experimental.pallas import tpu_sc as plsc

_N = 10000
_E = 320000
_D = 128

_NC = 2
_NS = 16
_NW = _NC * _NS
_EPW = _E // _NW
_CH = 80
_NCHUNK = _EPW // _CH
_RPT = _N // _NS

_mesh = plsc.VectorSubcoreMesh(core_axis_name="c", subcore_axis_name="s")


@functools.partial(
    pl.kernel,
    out_type=jax.ShapeDtypeStruct((_NC, 2, _N), jnp.float32),
    mesh=_mesh,
    scratch_types=[
        pltpu.VMEM((_CH,), jnp.int32),
        pltpu.VMEM((_CH,), jnp.int32),
        pltpu.VMEM((_CH,), jnp.float32),
        pltpu.VMEM((640,), jnp.float32),
        pltpu.VMEM_SHARED((_N,), jnp.float32),
        pltpu.VMEM_SHARED((_N,), jnp.float32),
    ],
)
def _deg_kernel(src_hbm, dst_hbm, out_hbm, sidx, didx, ones, zbuf, dego, degi):
    cid = lax.axis_index("c")
    sid = lax.axis_index("s")
    wid = sid * _NC + cid

    for j in range(_CH // 16):
        ones[pl.ds(j * 16, 16)] = jnp.ones((16,), jnp.float32)
    for j in range(640 // 16):
        zbuf[pl.ds(j * 16, 16)] = jnp.zeros((16,), jnp.float32)

    z0 = sid * 624
    pltpu.sync_copy(zbuf, dego.at[pl.ds(z0, 640)])
    pltpu.sync_copy(zbuf, degi.at[pl.ds(z0, 640)])
    plsc.subcore_barrier()

    base = wid * _EPW

    def body(i, carry):
        off = base + i * _CH
        pltpu.sync_copy(src_hbm.at[pl.ds(off, _CH)], sidx)
        pltpu.sync_copy(dst_hbm.at[pl.ds(off, _CH)], didx)
        pltpu.sync_copy(ones, dego.at[sidx], add=True)
        pltpu.sync_copy(ones, degi.at[didx], add=True)
        return carry

    lax.fori_loop(0, _NCHUNK, body, 0)
    plsc.subcore_barrier()

    @pl.when(sid == 0)
    def _():
        pltpu.sync_copy(dego, out_hbm.at[cid, 0])
        pltpu.sync_copy(degi, out_hbm.at[cid, 1])


@functools.partial(
    pl.kernel,
    out_type=jax.ShapeDtypeStruct((_NC, _N, _D), jnp.float32),
    mesh=_mesh,
    scratch_types=[
        pltpu.VMEM((_CH,), jnp.int32),
        pltpu.VMEM((_CH,), jnp.int32),
        pltpu.VMEM((_CH, _D), jnp.float32),
        pltpu.VMEM_SHARED((_N, _D), jnp.float32),
        pltpu.SemaphoreType.DMA,
    ],
)
def _agg_kernel(h_hbm, src_hbm, dst_hbm, out_hbm, sidx, didx, rows, acc, sem):
    cid = lax.axis_index("c")
    sid = lax.axis_index("s")
    wid = sid * _NC + cid

    def zrow(r, carry):
        for j in range(_D // 16):
            rows[r, pl.ds(j * 16, 16)] = jnp.zeros((16,), jnp.float32)
        return carry

    lax.fori_loop(0, _CH, zrow, 0)

    z0 = sid * 624
    for j in range(640 // _CH):
        pltpu.sync_copy(rows, acc.at[pl.ds(z0 + j * _CH, _CH)])
    plsc.subcore_barrier()

    base = wid * _EPW

    def body(i, carry):
        off = base + i * _CH
        pltpu.sync_copy(src_hbm.at[pl.ds(off, _CH)], sidx)
        pltpu.sync_copy(dst_hbm.at[pl.ds(off, _CH)], didx)
        pltpu.async_copy(h_hbm.at[sidx], rows, sem).wait()
        pltpu.sync_copy(rows, acc.at[didx], add=True)
        return carry

    lax.fori_loop(0, _NCHUNK, body, 0)
    plsc.subcore_barrier()

    pltpu.sync_copy(acc.at[pl.ds(z0, 640)],
                    out_hbm.at[cid, pl.ds(z0, 640)])


def _norm(deg):
    return lax.rsqrt(jnp.maximum(deg, 1.0))


def _l1_body(x_ref, dp_ref, w_ref, o_ref):
    ns = _norm(dp_ref[0, 0] + dp_ref[1, 0])
    o_ref[...] = jnp.dot(x_ref[...] * ns, w_ref[...],
                         preferred_element_type=jnp.float32)


def _l2_body(ap_ref, dp_ref, b_ref, w_ref, o_ref):
    agg = ap_ref[0] + ap_ref[1]
    nd = _norm(dp_ref[0, 1] + dp_ref[1, 1])
    ns = _norm(dp_ref[0, 0] + dp_ref[1, 0])
    h = jnp.maximum(agg * nd + b_ref[...], 0.0)
    o_ref[...] = jnp.dot(h * ns, w_ref[...],
                         preferred_element_type=jnp.float32)


def _l3_body(ap_ref, dp_ref, b_ref, o_ref):
    nd = _norm(dp_ref[0, 1] + dp_ref[1, 1])
    o_ref[...] = (ap_ref[0] + ap_ref[1]) * nd + b_ref[...]


_l1 = pl.pallas_call(
    _l1_body, out_shape=jax.ShapeDtypeStruct((_N, _D), jnp.float32))
_l2 = pl.pallas_call(
    _l2_body, out_shape=jax.ShapeDtypeStruct((_N, _D), jnp.float32))
_l3 = pl.pallas_call(
    _l3_body, out_shape=jax.ShapeDtypeStruct((_N, _D), jnp.float32))


def kernel(x, edge_index, W0, b0, W1, b1):
    src = edge_index[0]
    dst = edge_index[1]
    degp = _deg_kernel(src, dst)
    degp = degp.reshape(_NC, 2, _N, 1)
    h0 = _l1(x, degp, W0)
    aggp0 = _agg_kernel(h0, src, dst)
    h1 = _l2(aggp0, degp, b0.reshape(1, _D), W1)
    aggp1 = _agg_kernel(h1, src, dst)
    return _l3(aggp1, degp, b1.reshape(1, _D))


# speedup vs baseline: 12.2827x; 2.5365x over previous
import functools

import jax
import jax.numpy as jnp
from jax import lax
from jax.experimental import pallas as pl
from jax.experimental.pallas import tpu as pltpu
from jax.experimental.pallas import tpu_sc as plsc

_N = 10000
_E = 320000
_D = 128

_NC = 2
_NS = 16
_NW = _NC * _NS
_EPW = _E // _NW
_CH = 80
_NCHUNK = _EPW // _CH
_NBUF = 5
_NGRP = _NCHUNK // _NBUF

_mesh = plsc.VectorSubcoreMesh(core_axis_name="c", subcore_axis_name="s")


@functools.partial(
    pl.kernel,
    out_type=jax.ShapeDtypeStruct((_NC, 2, _N), jnp.float32),
    mesh=_mesh,
    scratch_types=[
        pltpu.VMEM((_NCHUNK, _CH), jnp.int32),
        pltpu.VMEM((_NCHUNK, _CH), jnp.int32),
        pltpu.VMEM((_CH,), jnp.float32),
        pltpu.VMEM((640,), jnp.float32),
        pltpu.VMEM_SHARED((_N,), jnp.float32),
        pltpu.VMEM_SHARED((_N,), jnp.float32),
        pltpu.SemaphoreType.DMA,
        pltpu.SemaphoreType.DMA,
    ],
)
def _deg_kernel(src_hbm, dst_hbm, out_hbm, sidx, didx, ones, zbuf,
                dego, degi, sem_s, sem_d):
    cid = lax.axis_index("c")
    sid = lax.axis_index("s")
    wid = sid * _NC + cid

    cps = pltpu.async_copy(src_hbm.at[wid], sidx, sem_s)
    cpd = pltpu.async_copy(dst_hbm.at[wid], didx, sem_d)

    for j in range(_CH // 16):
        ones[pl.ds(j * 16, 16)] = jnp.ones((16,), jnp.float32)
    for j in range(640 // 16):
        zbuf[pl.ds(j * 16, 16)] = jnp.zeros((16,), jnp.float32)

    z0 = sid * 624
    pltpu.sync_copy(zbuf, dego.at[pl.ds(z0, 640)])
    pltpu.sync_copy(zbuf, degi.at[pl.ds(z0, 640)])
    cps.wait()
    cpd.wait()
    plsc.subcore_barrier()

    _DEPTH = 8

    def fire(i):
        pltpu.async_copy(ones, dego.at[sidx.at[i]], sem_s, add=True)
        pltpu.async_copy(ones, degi.at[didx.at[i]], sem_d, add=True)

    def drain_one():
        pltpu.make_async_copy(ones, dego.at[sidx.at[0]], sem_s).wait()
        pltpu.make_async_copy(ones, degi.at[didx.at[0]], sem_d).wait()

    def head(i, carry):
        fire(i)
        return carry

    def steady(i, carry):
        fire(i)
        drain_one()
        return carry

    def tail(i, carry):
        drain_one()
        return carry

    lax.fori_loop(0, _DEPTH, head, 0)
    lax.fori_loop(_DEPTH, _NCHUNK, steady, 0)
    lax.fori_loop(0, _DEPTH, tail, 0)
    plsc.subcore_barrier()

    @pl.when(sid == 0)
    def _():
        pltpu.sync_copy(dego, out_hbm.at[cid, 0])
        pltpu.sync_copy(degi, out_hbm.at[cid, 1])


_RB = 2
_IQ = 6


@functools.partial(
    pl.kernel,
    out_type=jax.ShapeDtypeStruct((_NC, _N, _D), jnp.float32),
    mesh=_mesh,
    scratch_types=[
        [pltpu.VMEM((_CH,), jnp.int32)] * _IQ,
        [pltpu.VMEM((_CH,), jnp.int32)] * _IQ,
        [pltpu.VMEM((_CH, _D), jnp.float32)] * _RB,
        pltpu.VMEM_SHARED((_N, _D), jnp.float32),
        [pltpu.SemaphoreType.DMA] * _IQ,
        [pltpu.SemaphoreType.DMA] * _IQ,
        [pltpu.SemaphoreType.DMA] * _RB,
        [pltpu.SemaphoreType.DMA] * _RB,
    ],
)
def _agg_kernel(h_hbm, src_hbm, dst_hbm, out_hbm, sidx, didx, rings, acc,
                isems_s, isems_d, gsems, ssems):
    cid = lax.axis_index("c")
    sid = lax.axis_index("s")
    wid = sid * _NC + cid
    base = wid * _EPW

    def fire_idx(i, q):
        off = base + i * _CH
        pltpu.async_copy(src_hbm.at[pl.ds(off, _CH)], sidx[q], isems_s[q])
        pltpu.async_copy(dst_hbm.at[pl.ds(off, _CH)], didx[q], isems_d[q])

    def wait_idx_s(q):
        pltpu.make_async_copy(
            src_hbm.at[pl.ds(0, _CH)], sidx[q], isems_s[q]).wait()

    def wait_idx_d(q):
        pltpu.make_async_copy(
            dst_hbm.at[pl.ds(0, _CH)], didx[q], isems_d[q]).wait()

    for q in range(_IQ):
        fire_idx(q, q)

    def zrow(r, carry):
        for j in range(_D // 16):
            rings[0][r, pl.ds(j * 16, 16)] = jnp.zeros((16,), jnp.float32)
        return carry

    lax.fori_loop(0, _CH, zrow, 0)

    z0 = sid * 624
    for j in range(640 // _CH):
        pltpu.sync_copy(rings[0], acc.at[pl.ds(z0 + j * _CH, _CH)])
    plsc.subcore_barrier()

    for b in range(_RB):
        wait_idx_s(b)
        pltpu.async_copy(h_hbm.at[sidx[b]], rings[b], gsems[b])

    def step(i, b, q, refill_idx, regather):
        pltpu.make_async_copy(h_hbm.at[sidx[q]], rings[b], gsems[b]).wait()
        wait_idx_d(q)
        pltpu.async_copy(rings[b], acc.at[didx[q]], ssems[b], add=True)
        pltpu.make_async_copy(rings[b], acc.at[didx[q]], ssems[b]).wait()
        if refill_idx:
            fire_idx(i + _IQ, q)
        if regather:
            qn = (q + _RB) % _IQ
            wait_idx_s(qn)
            pltpu.async_copy(h_hbm.at[sidx[qn]], rings[b], gsems[b])

    def group(g, carry):
        for j in range(_IQ):
            i = g * _IQ + j
            step(i, j % _RB, j, True, True)
        return carry

    _NGRP6 = (_NCHUNK - _IQ) // _IQ
    lax.fori_loop(0, _NGRP6, group, 0)
    for i in range(_NGRP6 * _IQ, _NCHUNK):
        step(i, i % _RB, i % _IQ, i + _IQ < _NCHUNK, i + _RB < _NCHUNK)
    plsc.subcore_barrier()

    pltpu.sync_copy(acc.at[pl.ds(z0, 640)],
                    out_hbm.at[cid, pl.ds(z0, 640)])


def _norm(deg):
    return lax.rsqrt(jnp.maximum(deg, 1.0))


def _l1_body(x_ref, dp_ref, w_ref, o_ref):
    ns = _norm(dp_ref[0, 0] + dp_ref[1, 0])
    o_ref[...] = jnp.dot(x_ref[...] * ns, w_ref[...],
                         preferred_element_type=jnp.float32)


def _l2_body(ap_ref, dp_ref, b_ref, w_ref, o_ref):
    agg = ap_ref[0] + ap_ref[1]
    nd = _norm(dp_ref[0, 1] + dp_ref[1, 1])
    ns = _norm(dp_ref[0, 0] + dp_ref[1, 0])
    h = jnp.maximum(agg * nd + b_ref[...], 0.0)
    o_ref[...] = jnp.dot(h * ns, w_ref[...],
                         preferred_element_type=jnp.float32)


def _l3_body(ap_ref, dp_ref, b_ref, o_ref):
    nd = _norm(dp_ref[0, 1] + dp_ref[1, 1])
    o_ref[...] = (ap_ref[0] + ap_ref[1]) * nd + b_ref[...]


_l1 = pl.pallas_call(
    _l1_body, out_shape=jax.ShapeDtypeStruct((_N, _D), jnp.float32))
_l2 = pl.pallas_call(
    _l2_body, out_shape=jax.ShapeDtypeStruct((_N, _D), jnp.float32))
_l3 = pl.pallas_call(
    _l3_body, out_shape=jax.ShapeDtypeStruct((_N, _D), jnp.float32))


def kernel(x, edge_index, W0, b0, W1, b1):
    src = edge_index[0]
    dst = edge_index[1]
    src3 = src.reshape(_NW, _NCHUNK, _CH)
    dst3 = dst.reshape(_NW, _NCHUNK, _CH)
    degp = _deg_kernel(src3, dst3)
    degp = degp.reshape(_NC, 2, _N, 1)
    h0 = _l1(x, degp, W0)
    aggp0 = _agg_kernel(h0, src, dst)
    h1 = _l2(aggp0, degp, b0.reshape(1, _D), W1)
    aggp1 = _agg_kernel(h1, src, dst)
    return _l3(aggp1, degp, b1.reshape(1, _D))


# speedup vs baseline: 14.2070x; 1.1567x over previous
import functools

import jax
import jax.numpy as jnp
from jax import lax
from jax.experimental import pallas as pl
from jax.experimental.pallas import tpu as pltpu
from jax.experimental.pallas import tpu_sc as plsc

_N = 10000
_E = 320000
_D = 128

_NC = 2
_NS = 16
_NW = _NC * _NS
_EPW = _E // _NW
_CH = 80
_NCHUNK = _EPW // _CH
_NBUF = 5
_NGRP = _NCHUNK // _NBUF

_mesh = plsc.VectorSubcoreMesh(core_axis_name="c", subcore_axis_name="s")


@functools.partial(
    pl.kernel,
    out_type=jax.ShapeDtypeStruct((_NC, 2, _N), jnp.float32),
    mesh=_mesh,
    scratch_types=[
        pltpu.VMEM((_NCHUNK, _CH), jnp.int32),
        pltpu.VMEM((_NCHUNK, _CH), jnp.int32),
        pltpu.VMEM((_CH,), jnp.float32),
        pltpu.VMEM((640,), jnp.float32),
        pltpu.VMEM_SHARED((_N,), jnp.float32),
        pltpu.VMEM_SHARED((_N,), jnp.float32),
        pltpu.SemaphoreType.DMA,
        pltpu.SemaphoreType.DMA,
    ],
)
def _deg_kernel(src_hbm, dst_hbm, out_hbm, sidx, didx, ones, zbuf,
                dego, degi, sem_s, sem_d):
    cid = lax.axis_index("c")
    sid = lax.axis_index("s")
    wid = sid * _NC + cid

    cps = pltpu.async_copy(src_hbm.at[wid], sidx, sem_s)
    cpd = pltpu.async_copy(dst_hbm.at[wid], didx, sem_d)

    for j in range(_CH // 16):
        ones[pl.ds(j * 16, 16)] = jnp.ones((16,), jnp.float32)
    for j in range(640 // 16):
        zbuf[pl.ds(j * 16, 16)] = jnp.zeros((16,), jnp.float32)

    z0 = sid * 624
    pltpu.sync_copy(zbuf, dego.at[pl.ds(z0, 640)])
    pltpu.sync_copy(zbuf, degi.at[pl.ds(z0, 640)])
    cps.wait()
    cpd.wait()
    plsc.subcore_barrier()

    _DEPTH = 8

    def fire(i):
        pltpu.async_copy(ones, dego.at[sidx.at[i]], sem_s, add=True)
        pltpu.async_copy(ones, degi.at[didx.at[i]], sem_d, add=True)

    def drain_one():
        pltpu.make_async_copy(ones, dego.at[sidx.at[0]], sem_s).wait()
        pltpu.make_async_copy(ones, degi.at[didx.at[0]], sem_d).wait()

    def head(i, carry):
        fire(i)
        return carry

    def steady(i, carry):
        fire(i)
        drain_one()
        return carry

    def tail(i, carry):
        drain_one()
        return carry

    lax.fori_loop(0, _DEPTH, head, 0)
    lax.fori_loop(_DEPTH, _NCHUNK, steady, 0)
    lax.fori_loop(0, _DEPTH, tail, 0)
    plsc.subcore_barrier()

    @pl.when(sid == 0)
    def _():
        pltpu.sync_copy(dego, out_hbm.at[cid, 0])
        pltpu.sync_copy(degi, out_hbm.at[cid, 1])


_RB = 3
_IQ = 6


@functools.partial(
    pl.kernel,
    out_type=jax.ShapeDtypeStruct((_NC, _N, _D), jnp.float32),
    mesh=_mesh,
    scratch_types=[
        [pltpu.VMEM((_CH,), jnp.int32)] * _IQ,
        [pltpu.VMEM((_CH,), jnp.int32)] * _IQ,
        [pltpu.VMEM((_CH, _D), jnp.float32)] * _RB,
        pltpu.VMEM_SHARED((_N, _D), jnp.float32),
        [pltpu.SemaphoreType.DMA] * _IQ,
        [pltpu.SemaphoreType.DMA] * _IQ,
        [pltpu.SemaphoreType.DMA] * _RB,
        [pltpu.SemaphoreType.DMA] * _RB,
    ],
)
def _agg_kernel(h_hbm, src_hbm, dst_hbm, out_hbm, sidx, didx, rings, acc,
                isems_s, isems_d, gsems, ssems):
    cid = lax.axis_index("c")
    sid = lax.axis_index("s")
    wid = sid * _NC + cid
    base = wid * _EPW

    def fire_idx(i, q):
        off = base + i * _CH
        pltpu.async_copy(src_hbm.at[pl.ds(off, _CH)], sidx[q], isems_s[q])
        pltpu.async_copy(dst_hbm.at[pl.ds(off, _CH)], didx[q], isems_d[q])

    def wait_idx_s(q):
        pltpu.make_async_copy(
            src_hbm.at[pl.ds(0, _CH)], sidx[q], isems_s[q]).wait()

    def wait_idx_d(q):
        pltpu.make_async_copy(
            dst_hbm.at[pl.ds(0, _CH)], didx[q], isems_d[q]).wait()

    for q in range(_IQ):
        fire_idx(q, q)

    def zrow(r, carry):
        for j in range(_D // 16):
            rings[0][r, pl.ds(j * 16, 16)] = jnp.zeros((16,), jnp.float32)
        return carry

    lax.fori_loop(0, _CH, zrow, 0)

    z0 = sid * 624
    for j in range(640 // _CH):
        pltpu.sync_copy(rings[0], acc.at[pl.ds(z0 + j * _CH, _CH)])
    plsc.subcore_barrier()

    for b in range(_RB):
        wait_idx_s(b)
        pltpu.async_copy(h_hbm.at[sidx[b]], rings[b], gsems[b])

    def step(i, b, q, refill_idx, regather):
        pltpu.make_async_copy(h_hbm.at[sidx[q]], rings[b], gsems[b]).wait()
        wait_idx_d(q)
        pltpu.async_copy(rings[b], acc.at[didx[q]], ssems[b], add=True)
        pltpu.make_async_copy(rings[b], acc.at[didx[q]], ssems[b]).wait()
        if refill_idx:
            fire_idx(i + _IQ, q)
        if regather:
            qn = (q + _RB) % _IQ
            wait_idx_s(qn)
            pltpu.async_copy(h_hbm.at[sidx[qn]], rings[b], gsems[b])

    def group(g, carry):
        for j in range(_IQ):
            i = g * _IQ + j
            step(i, j % _RB, j, True, True)
        return carry

    _NGRP6 = (_NCHUNK - _IQ) // _IQ
    lax.fori_loop(0, _NGRP6, group, 0)
    for i in range(_NGRP6 * _IQ, _NCHUNK):
        step(i, i % _RB, i % _IQ, i + _IQ < _NCHUNK, i + _RB < _NCHUNK)
    plsc.subcore_barrier()

    pltpu.sync_copy(acc.at[pl.ds(z0, 640)],
                    out_hbm.at[cid, pl.ds(z0, 640)])


def _norm(deg):
    return lax.rsqrt(jnp.maximum(deg, 1.0))


def _l1_body(x_ref, dp_ref, w_ref, o_ref):
    ns = _norm(dp_ref[0, 0] + dp_ref[1, 0])
    o_ref[...] = jnp.dot(x_ref[...] * ns, w_ref[...],
                         preferred_element_type=jnp.float32)


def _l2_body(ap_ref, dp_ref, b_ref, w_ref, o_ref):
    agg = ap_ref[0] + ap_ref[1]
    nd = _norm(dp_ref[0, 1] + dp_ref[1, 1])
    ns = _norm(dp_ref[0, 0] + dp_ref[1, 0])
    h = jnp.maximum(agg * nd + b_ref[...], 0.0)
    o_ref[...] = jnp.dot(h * ns, w_ref[...],
                         preferred_element_type=jnp.float32)


def _l3_body(ap_ref, dp_ref, b_ref, o_ref):
    nd = _norm(dp_ref[0, 1] + dp_ref[1, 1])
    o_ref[...] = (ap_ref[0] + ap_ref[1]) * nd + b_ref[...]


_l1 = pl.pallas_call(
    _l1_body, out_shape=jax.ShapeDtypeStruct((_N, _D), jnp.float32))
_l2 = pl.pallas_call(
    _l2_body, out_shape=jax.ShapeDtypeStruct((_N, _D), jnp.float32))
_l3 = pl.pallas_call(
    _l3_body, out_shape=jax.ShapeDtypeStruct((_N, _D), jnp.float32))


def kernel(x, edge_index, W0, b0, W1, b1):
    src = edge_index[0]
    dst = edge_index[1]
    src3 = src.reshape(_NW, _NCHUNK, _CH)
    dst3 = dst.reshape(_NW, _NCHUNK, _CH)
    degp = _deg_kernel(src3, dst3)
    degp = degp.reshape(_NC, 2, _N, 1)
    h0 = _l1(x, degp, W0)
    aggp0 = _agg_kernel(h0, src, dst)
    h1 = _l2(aggp0, degp, b0.reshape(1, _D), W1)
    aggp1 = _agg_kernel(h1, src, dst)
    return _l3(aggp1, degp, b1.reshape(1, _D))


# speedup vs baseline: 14.5682x; 1.0254x over previous
import functools
import math

import jax
import jax.numpy as jnp
from jax import lax
from jax.experimental import pallas as pl
from jax.experimental.pallas import tpu as pltpu
from jax.experimental.pallas import tpu_sc as plsc

_N = 10000
_E = 320000
_D = 128

_NC = 2
_NS = 16
_NW = _NC * _NS
_EPW = _E // _NW
_CH = 80
_NCHUNK = _EPW // _CH
_NBUF = 5
_NGRP = _NCHUNK // _NBUF

_mesh = plsc.VectorSubcoreMesh(core_axis_name="c", subcore_axis_name="s")


@functools.partial(
    pl.kernel,
    out_type=jax.ShapeDtypeStruct((_NC, 2, _N), jnp.float32),
    mesh=_mesh,
    scratch_types=[
        pltpu.VMEM((_NCHUNK, _CH), jnp.int32),
        pltpu.VMEM((_NCHUNK, _CH), jnp.int32),
        pltpu.VMEM((_CH,), jnp.float32),
        pltpu.VMEM((640,), jnp.float32),
        pltpu.VMEM_SHARED((_N,), jnp.float32),
        pltpu.VMEM_SHARED((_N,), jnp.float32),
        pltpu.SemaphoreType.DMA,
        pltpu.SemaphoreType.DMA,
    ],
)
def _deg_kernel(src_hbm, dst_hbm, out_hbm, sidx, didx, ones, zbuf,
                dego, degi, sem_s, sem_d):
    cid = lax.axis_index("c")
    sid = lax.axis_index("s")
    wid = sid * _NC + cid

    cps = pltpu.async_copy(src_hbm.at[wid], sidx, sem_s)
    cpd = pltpu.async_copy(dst_hbm.at[wid], didx, sem_d)

    for j in range(_CH // 16):
        ones[pl.ds(j * 16, 16)] = jnp.ones((16,), jnp.float32)
    for j in range(640 // 16):
        zbuf[pl.ds(j * 16, 16)] = jnp.zeros((16,), jnp.float32)

    z0 = sid * 624
    pltpu.sync_copy(zbuf, dego.at[pl.ds(z0, 640)])
    pltpu.sync_copy(zbuf, degi.at[pl.ds(z0, 640)])
    cps.wait()
    cpd.wait()
    plsc.subcore_barrier()

    _DEPTH = 8

    def fire(i):
        pltpu.async_copy(ones, dego.at[sidx.at[i]], sem_s, add=True)
        pltpu.async_copy(ones, degi.at[didx.at[i]], sem_d, add=True)

    def drain_one():
        pltpu.make_async_copy(ones, dego.at[sidx.at[0]], sem_s).wait()
        pltpu.make_async_copy(ones, degi.at[didx.at[0]], sem_d).wait()

    def head(i, carry):
        fire(i)
        return carry

    def steady(i, carry):
        fire(i)
        drain_one()
        return carry

    def tail(i, carry):
        drain_one()
        return carry

    lax.fori_loop(0, _DEPTH, head, 0)
    lax.fori_loop(_DEPTH, _NCHUNK, steady, 0)
    lax.fori_loop(0, _DEPTH, tail, 0)
    plsc.subcore_barrier()

    @pl.when(sid == 0)
    def _():
        pltpu.sync_copy(dego, out_hbm.at[cid, 0])
        pltpu.sync_copy(degi, out_hbm.at[cid, 1])


_RB = 4
_IQ = 6
_GRP = math.lcm(_RB, _IQ)


@functools.partial(
    pl.kernel,
    out_type=jax.ShapeDtypeStruct((_NC, _N, _D), jnp.float32),
    mesh=_mesh,
    scratch_types=[
        [pltpu.VMEM((_CH,), jnp.int32)] * _IQ,
        [pltpu.VMEM((_CH,), jnp.int32)] * _IQ,
        [pltpu.VMEM((_CH, _D), jnp.float32)] * _RB,
        pltpu.VMEM_SHARED((_N, _D), jnp.float32),
        [pltpu.SemaphoreType.DMA] * _IQ,
        [pltpu.SemaphoreType.DMA] * _IQ,
        [pltpu.SemaphoreType.DMA] * _RB,
        [pltpu.SemaphoreType.DMA] * _RB,
    ],
)
def _agg_kernel(h_hbm, src_hbm, dst_hbm, out_hbm, sidx, didx, rings, acc,
                isems_s, isems_d, gsems, ssems):
    cid = lax.axis_index("c")
    sid = lax.axis_index("s")
    wid = sid * _NC + cid
    base = wid * _EPW

    def fire_idx(i, q):
        off = base + i * _CH
        pltpu.async_copy(src_hbm.at[pl.ds(off, _CH)], sidx[q], isems_s[q])
        pltpu.async_copy(dst_hbm.at[pl.ds(off, _CH)], didx[q], isems_d[q])

    def wait_idx_s(q):
        pltpu.make_async_copy(
            src_hbm.at[pl.ds(0, _CH)], sidx[q], isems_s[q]).wait()

    def wait_idx_d(q):
        pltpu.make_async_copy(
            dst_hbm.at[pl.ds(0, _CH)], didx[q], isems_d[q]).wait()

    for q in range(_IQ):
        fire_idx(q, q)

    def zrow(r, carry):
        for j in range(_D // 16):
            rings[0][r, pl.ds(j * 16, 16)] = jnp.zeros((16,), jnp.float32)
        return carry

    lax.fori_loop(0, _CH, zrow, 0)

    z0 = sid * 624
    for j in range(640 // _CH):
        pltpu.sync_copy(rings[0], acc.at[pl.ds(z0 + j * _CH, _CH)])
    plsc.subcore_barrier()

    for b in range(_RB):
        wait_idx_s(b)
        pltpu.async_copy(h_hbm.at[sidx[b]], rings[b], gsems[b])

    def step(i, b, q, refill_idx, regather):
        pltpu.make_async_copy(h_hbm.at[sidx[q]], rings[b], gsems[b]).wait()
        wait_idx_d(q)
        pltpu.async_copy(rings[b], acc.at[didx[q]], ssems[b], add=True)
        pltpu.make_async_copy(rings[b], acc.at[didx[q]], ssems[b]).wait()
        if refill_idx:
            fire_idx(i + _IQ, q)
        if regather:
            qn = (q + _RB) % _IQ
            wait_idx_s(qn)
            pltpu.async_copy(h_hbm.at[sidx[qn]], rings[b], gsems[b])

    def group(g, carry):
        for j in range(_GRP):
            i = g * _GRP + j
            step(i, j % _RB, j % _IQ, True, True)
        return carry

    _NSTEADY = (_NCHUNK - _IQ) // _GRP
    lax.fori_loop(0, _NSTEADY, group, 0)
    for i in range(_NSTEADY * _GRP, _NCHUNK):
        step(i, i % _RB, i % _IQ, i + _IQ < _NCHUNK, i + _RB < _NCHUNK)
    plsc.subcore_barrier()

    pltpu.sync_copy(acc.at[pl.ds(z0, 640)],
                    out_hbm.at[cid, pl.ds(z0, 640)])


def _norm(deg):
    return lax.rsqrt(jnp.maximum(deg, 1.0))


def _l1_body(x_ref, dp_ref, w_ref, o_ref):
    ns = _norm(dp_ref[0, 0] + dp_ref[1, 0])
    o_ref[...] = jnp.dot(x_ref[...] * ns, w_ref[...],
                         preferred_element_type=jnp.float32)


def _l2_body(ap_ref, dp_ref, b_ref, w_ref, o_ref):
    agg = ap_ref[0] + ap_ref[1]
    nd = _norm(dp_ref[0, 1] + dp_ref[1, 1])
    ns = _norm(dp_ref[0, 0] + dp_ref[1, 0])
    h = jnp.maximum(agg * nd + b_ref[...], 0.0)
    o_ref[...] = jnp.dot(h * ns, w_ref[...],
                         preferred_element_type=jnp.float32)


def _l3_body(ap_ref, dp_ref, b_ref, o_ref):
    nd = _norm(dp_ref[0, 1] + dp_ref[1, 1])
    o_ref[...] = (ap_ref[0] + ap_ref[1]) * nd + b_ref[...]


_l1 = pl.pallas_call(
    _l1_body, out_shape=jax.ShapeDtypeStruct((_N, _D), jnp.float32))
_l2 = pl.pallas_call(
    _l2_body, out_shape=jax.ShapeDtypeStruct((_N, _D), jnp.float32))
_l3 = pl.pallas_call(
    _l3_body, out_shape=jax.ShapeDtypeStruct((_N, _D), jnp.float32))


def kernel(x, edge_index, W0, b0, W1, b1):
    src = edge_index[0]
    dst = edge_index[1]
    src3 = src.reshape(_NW, _NCHUNK, _CH)
    dst3 = dst.reshape(_NW, _NCHUNK, _CH)
    degp = _deg_kernel(src3, dst3)
    degp = degp.reshape(_NC, 2, _N, 1)
    h0 = _l1(x, degp, W0)
    aggp0 = _agg_kernel(h0, src, dst)
    h1 = _l2(aggp0, degp, b0.reshape(1, _D), W1)
    aggp1 = _agg_kernel(h1, src, dst)
    return _l3(aggp1, degp, b1.reshape(1, _D))
